# Initial kernel scaffold; baseline (speedup 1.0000x reference)
import jax
import jax.numpy as jnp
from jax.experimental import pallas as pl


def kernel(x, edge_index, edge_type, edge_attr, future_return, W1, q1, k1, e1, b1, bn_gamma, bn_beta, W2, q2, k2, e2, b2):
    raise NotImplementedError("write your pallas kernel here")


# speedup vs baseline: 88.4585x; 88.4585x over previous
import dataclasses
import functools

import jax
import jax.numpy as jnp
from jax import lax
from jax.experimental import pallas as pl
from jax.experimental.pallas import tpu as pltpu
from jax.experimental.pallas import tpu_sc as plsc

N = 10000
E = 320000
F = 128
H1 = 10
R = 4

NW = 32
CH = 128
EPW = 10112
EP = NW * EPW
NCHUNK = EPW // CH
NP = 10240
TP = 40960

_f32 = jnp.float32
_i32 = jnp.int32


def _mm_body(x_ref, w_ref, o_ref):
    o_ref[...] = lax.dot_general(
        x_ref[...], w_ref[...], (((1,), (0,)), ((), ())),
        precision=lax.Precision.HIGHEST)


def _node_body(p_ref, cb_ref, w2_ref, o_ref):
    s = p_ref[0] + p_ref[1]
    cb = cb_ref[...]
    e1v, b1v, gam, bet = cb[0:1], cb[1:2], cb[2:3], cb[3:4]
    den = s[:, 10:11]
    sa = s[:, 11:12]
    out1 = (s + sa * e1v) / (den + 1e-16) + b1v
    v = out1[0:N]
    mu = jnp.mean(v, axis=0, keepdims=True)
    var = jnp.mean((v - mu) ** 2, axis=0, keepdims=True)
    h = (v - mu) / jnp.sqrt(var + 1e-5) * gam + bet
    h = jnp.where(h > 0.0, h, jnp.exp(h) - 1.0)
    o_ref[...] = lax.dot_general(
        h, w2_ref[...], (((1,), (0,)), ((), ())),
        precision=lax.Precision.HIGHEST)


def _resid_body(d_ref, fr_ref, b2_ref, o_ref):
    den = d_ref[0:1] + d_ref[2:3]
    num = d_ref[1:2] + d_ref[3:4]
    y = num / (den + 1e-16) + b2_ref[0, 0]
    o_ref[...] = fr_ref[...] - y


_MESH = plsc.VectorSubcoreMesh(core_axis_name="c", subcore_axis_name="s")

_CP = pltpu.CompilerParams()
if "needs_layout_passes" in pltpu.CompilerParams.__dataclass_fields__:
    _CP = dataclasses.replace(_CP, needs_layout_passes=False,
                              use_tc_tiling_on_sc=False)


@functools.partial(
    pl.kernel,
    mesh=_MESH,
    compiler_params=_CP,
    out_type=jax.ShapeDtypeStruct((2, NP, 16), _f32),
    scratch_types=[
        pltpu.VMEM((TP,), _f32),
        pltpu.VMEM((TP,), _f32),
        pltpu.VMEM((8,), _f32),
        pltpu.VMEM((4, CH), _i32),
        pltpu.VMEM((CH,), _i32),
        pltpu.VMEM((CH,), _i32),
        pltpu.VMEM((CH,), _f32),
        pltpu.VMEM((CH,), _f32),
        pltpu.VMEM((CH, 16), _f32),
        pltpu.VMEM((NP // 16, 16), _f32),
        pltpu.VMEM_SHARED((NP, 16), _f32),
        pltpu.SemaphoreType.DMA,
    ],
)
def _sc_edge1(edata, q1f, k1f, c1t, m64, out,
              q_tab, k_tab, c_tab, ed, d_idx, j_idx, exb, exab, rows,
              zbuf, num_sh, sem):
    cid = lax.axis_index("c")
    sid = lax.axis_index("s")
    w = sid * 2 + cid

    cp1 = pltpu.async_copy(q1f, q_tab, sem)
    cp2 = pltpu.async_copy(k1f, k_tab, sem)
    cp3 = pltpu.async_copy(c1t, c_tab, sem)

    zrow = jnp.zeros((16,), _f32)

    @pl.loop(0, NP // 16)
    def _(i):
        zbuf[i] = zrow

    pltpu.sync_copy(zbuf, num_sh.at[pl.ds(sid * (NP // 16), NP // 16)])
    cp1.wait()
    cp2.wait()
    cp3.wait()
    plsc.subcore_barrier()

    lanes = lax.iota(_i32, 16)
    base0 = w * EPW

    @pl.loop(0, NCHUNK)
    def _(c):
        base = base0 + c * CH
        pltpu.sync_copy(edata.at[:, pl.ds(base, CH)], ed)
        for i in range(CH // 16):
            sl = pl.ds(i * 16, 16)
            s = ed[0, sl]
            d = ed[1, sl]
            t = ed[2, sl]
            ab = plsc.bitcast(ed[3, sl], _f32)
            jq = d * R + t
            jk = s * R + t
            qv = plsc.load_gather(q_tab, [jq])
            kv = plsc.load_gather(k_tab, [jk])
            cv = plsc.load_gather(c_tab, [t])
            z = qv + kv + ab * cv
            z = jnp.where(z >= 0.0, z, 0.2 * z)
            ex = jnp.exp(z)
            j_idx[sl] = jk
            d_idx[sl] = d
            exb[sl] = ex
            exab[sl] = ex * ab
        pltpu.sync_copy(m64.at[j_idx], rows)

        @pl.loop(0, CH // 16)
        def _(i):
            exv = exb[pl.ds(i * 16, 16)]
            eav = exab[pl.ds(i * 16, 16)]
            for j in range(16):
                e = exv[j]
                ea = eav[j]
                row = rows[i * 16 + j] * e
                row = jnp.where(lanes == 10, e, row)
                row = jnp.where(lanes == 11, ea, row)
                rows[i * 16 + j] = row

        pltpu.sync_copy(rows, num_sh.at[d_idx], add=True)

    plsc.subcore_barrier()

    @pl.when(sid == 0)
    def _():
        pltpu.sync_copy(num_sh, out.at[cid])


@functools.partial(
    pl.kernel,
    mesh=_MESH,
    compiler_params=_CP,
    out_type=jax.ShapeDtypeStruct((2, 2, NP), _f32),
    scratch_types=[
        pltpu.VMEM((TP,), _f32),
        pltpu.VMEM((8,), _f32),
        pltpu.VMEM((8,), _f32),
        pltpu.VMEM((8,), _f32),
        pltpu.VMEM((16,), _f32),
        pltpu.VMEM((4, CH), _i32),
        pltpu.VMEM((CH,), _i32),
        pltpu.VMEM((CH,), _f32),
        pltpu.VMEM((CH,), _f32),
        pltpu.VMEM((NP // 16,), _f32),
        pltpu.VMEM_SHARED((NP,), _f32),
        pltpu.VMEM_SHARED((NP,), _f32),
        pltpu.SemaphoreType.DMA,
    ],
)
def _sc_edge2(edata, x2f, q2v, k2v, c2v, miscv, out,
              x2_tab, q2t, k2t, c2t, misc, ed, d_idx, exb, mb,
              zbuf, den_sh, num_sh, sem):
    cid = lax.axis_index("c")
    sid = lax.axis_index("s")
    w = sid * 2 + cid

    cp1 = pltpu.async_copy(x2f, x2_tab, sem)
    cp2 = pltpu.async_copy(q2v, q2t, sem)
    cp3 = pltpu.async_copy(k2v, k2t, sem)
    cp4 = pltpu.async_copy(c2v, c2t, sem)
    cp5 = pltpu.async_copy(miscv, misc, sem)

    zv = jnp.zeros((16,), _f32)

    @pl.loop(0, NP // 256)
    def _(i):
        zbuf[pl.ds(i * 16, 16)] = zv

    stripe = pl.ds(sid * (NP // 16), NP // 16)
    pltpu.sync_copy(zbuf, den_sh.at[stripe])
    pltpu.sync_copy(zbuf, num_sh.at[stripe])
    cp1.wait()
    cp2.wait()
    cp3.wait()
    cp4.wait()
    cp5.wait()
    plsc.subcore_barrier()

    e2s = misc[pl.ds(0, 16)]
    base0 = w * EPW

    @pl.loop(0, NCHUNK)
    def _(c):
        base = base0 + c * CH
        pltpu.sync_copy(edata.at[:, pl.ds(base, CH)], ed)
        for i in range(CH // 16):
            sl = pl.ds(i * 16, 16)
            s = ed[0, sl]
            d = ed[1, sl]
            t = ed[2, sl]
            ab = plsc.bitcast(ed[3, sl], _f32)
            jq = d * R + t
            jk = s * R + t
            av = plsc.load_gather(x2_tab, [jq])
            bv = plsc.load_gather(x2_tab, [jk])
            qt = plsc.load_gather(q2t, [t])
            kt = plsc.load_gather(k2t, [t])
            ct = plsc.load_gather(c2t, [t])
            z = av * qt + bv * kt + ab * ct
            z = jnp.where(z >= 0.0, z, 0.2 * z)
            ex = jnp.exp(z)
            d_idx[sl] = d
            exb[sl] = ex
            mb[sl] = ex * (bv + ab * e2s)
        pltpu.sync_copy(exb, den_sh.at[d_idx], add=True)
        pltpu.sync_copy(mb, num_sh.at[d_idx], add=True)

    plsc.subcore_barrier()

    @pl.when(sid == 0)
    def _():
        pltpu.sync_copy(den_sh, out.at[cid, 0])
        pltpu.sync_copy(num_sh, out.at[cid, 1])


def kernel(x, edge_index, edge_type, edge_attr, future_return,
           W1, q1, k1, e1, b1, bn_gamma, bn_beta, W2, q2, k2, e2, b2):
    w1t = jnp.transpose(W1, (1, 0, 2))
    w64 = jnp.pad(w1t, ((0, 0), (0, 0), (0, 16 - H1))).reshape(F, R * 16)
    wq = jnp.einsum('tio,to->it', W1, q1)
    wk = jnp.einsum('tio,to->it', W1, k1)
    wcat = jnp.concatenate([w64, wq, wk], axis=1)
    c1 = jnp.pad(jnp.einsum('o,to->t', e1[0], k1), (0, 4))

    w2m = jnp.pad(jnp.transpose(W2[:, :, 0]), ((0, 6), (0, 4)))
    q2t = jnp.pad(q2[:, 0], (0, 4))
    k2t = jnp.pad(k2[:, 0], (0, 4))
    c2t = jnp.pad(e2[0, 0] * k2[:, 0], (0, 4))
    misc = jnp.full((16,), e2[0, 0], _f32)

    cb = jnp.zeros((8, 16), _f32)
    cb = cb.at[0, :H1].set(e1[0])
    cb = cb.at[1, :H1].set(b1)
    cb = cb.at[2, :H1].set(bn_gamma)
    cb = cb.at[3, :H1].set(bn_beta)

    pad = EP - E
    src_p = jnp.pad(edge_index[0], (0, pad))
    dst_p = jnp.pad(edge_index[1], (0, pad), constant_values=N)
    et_p = jnp.pad(edge_type, (0, pad))
    ab_p = lax.bitcast_convert_type(jnp.pad(edge_attr[:, 0], (0, pad)), _i32)
    edata = jnp.stack([src_p, dst_p, et_p, ab_p])

    u = pl.pallas_call(
        _mm_body,
        grid=(10,),
        in_specs=[pl.BlockSpec((N // 10, F), lambda i: (i, 0)),
                  pl.BlockSpec((F, 72), lambda i: (0, 0))],
        out_specs=pl.BlockSpec((N // 10, 72), lambda i: (i, 0)),
        out_shape=jax.ShapeDtypeStruct((N, 72), _f32),
    )(x, wcat)
    m64 = u[:, :64].reshape(N * R, 16)
    q1f = jnp.pad(u[:, 64:68].reshape(-1), (0, TP - N * R))
    k1f = jnp.pad(u[:, 68:72].reshape(-1), (0, TP - N * R))

    p1 = _sc_edge1(edata, q1f, k1f, c1, m64)

    u2 = pl.pallas_call(
        _node_body,
        out_shape=jax.ShapeDtypeStruct((N, 8), _f32),
    )(p1, cb, w2m)
    x2f = jnp.pad(u2[:, :R].reshape(-1), (0, TP - N * R))

    p2 = _sc_edge2(edata, x2f, q2t, k2t, c2t, misc)

    frp = jnp.pad(future_return, (0, NP - N)).reshape(1, NP)
    res = pl.pallas_call(
        _resid_body,
        out_shape=jax.ShapeDtypeStruct((1, NP), _f32),
    )(p2.reshape(4, NP), frp, b2.reshape(1, 1))
    return res.reshape(NP)[:N]
